# baseline (speedup 1.0000x reference)
import jax
import jax.numpy as jnp
from jax.experimental import pallas as pl
from jax.experimental.pallas import tpu as pltpu

_B, _D, _E, _H, _C = 2048, 1024, 16, 256, 100
_CP = 128
_TB = 512


def _fused_kernel(x_ref, proto_ref, W1_ref, b1_ref, g_ref, bb_ref,
                  W2_ref, b2_ref, out_ref, w_scratch):
    e = pl.program_id(1)
    xb = x_ref[...]

    @pl.when(e == 0)
    def _compute_router():
        xf = xb.astype(jnp.float32)
        xn = xf / (jnp.sqrt(jnp.sum(xf * xf, axis=1, keepdims=True)) + 1e-8)
        p = proto_ref[...]
        pn = p / (jnp.sqrt(jnp.sum(p * p, axis=1, keepdims=True)) + 1e-8)
        sims = jnp.dot(xn, pn.T, preferred_element_type=jnp.float32)
        w_scratch[...] = jax.nn.softmax(sims, axis=-1)

    h = jnp.dot(xb, W1_ref[0], preferred_element_type=jnp.float32) + b1_ref[0]
    mu = jnp.mean(h, axis=-1, keepdims=True)
    var = jnp.mean((h - mu) ** 2, axis=-1, keepdims=True)
    h = (h - mu) / jnp.sqrt(var + 1e-5)
    h = h * g_ref[0] + bb_ref[0]
    h = jax.nn.gelu(h).astype(jnp.bfloat16)
    logits = (jnp.dot(h, W2_ref[0], preferred_element_type=jnp.float32)
              + b2_ref[0])

    w = w_scratch[...]
    lane = jax.lax.broadcasted_iota(jnp.int32, w.shape, 1)
    w_col = jnp.sum(jnp.where(lane == e, w, 0.0), axis=1, keepdims=True)

    @pl.when(e == 0)
    def _init():
        out_ref[...] = w_col * logits

    @pl.when(e != 0)
    def _acc():
        out_ref[...] += w_col * logits


@jax.jit
def kernel(x, prototypes, W1, b1, ln_g, ln_b, W2, b2):
    x16 = x.astype(jnp.bfloat16)
    W1b = W1.astype(jnp.bfloat16)
    W2p = jnp.pad(W2, ((0, 0), (0, 0), (0, _CP - _C))).astype(jnp.bfloat16)
    b2p = jnp.pad(b2, ((0, 0), (0, _CP - _C)))
    b1r = b1.reshape(_E, 1, _H)
    gr = ln_g.reshape(_E, 1, _H)
    br = ln_b.reshape(_E, 1, _H)
    b2r = b2p.reshape(_E, 1, _CP)
    nb = _B // _TB
    out = pl.pallas_call(
        _fused_kernel,
        grid=(nb, _E),
        in_specs=[
            pl.BlockSpec((_TB, _D), lambda b, e: (b, 0)),
            pl.BlockSpec((_E, _D), lambda b, e: (0, 0)),
            pl.BlockSpec((1, _D, _H), lambda b, e: (e, 0, 0)),
            pl.BlockSpec((1, 1, _H), lambda b, e: (e, 0, 0)),
            pl.BlockSpec((1, 1, _H), lambda b, e: (e, 0, 0)),
            pl.BlockSpec((1, 1, _H), lambda b, e: (e, 0, 0)),
            pl.BlockSpec((1, _H, _CP), lambda b, e: (e, 0, 0)),
            pl.BlockSpec((1, 1, _CP), lambda b, e: (e, 0, 0)),
        ],
        out_specs=pl.BlockSpec((_TB, _CP), lambda b, e: (b, 0)),
        out_shape=jax.ShapeDtypeStruct((_B, _CP), jnp.float32),
        scratch_shapes=[pltpu.VMEM((_TB, _E), jnp.float32)],
        compiler_params=pltpu.CompilerParams(
            dimension_semantics=("parallel", "arbitrary")),
    )(x16, prototypes, W1b, b1r, gr, br, W2p, b2r)
    return out[:, :_C]


# speedup vs baseline: 1.8128x; 1.8128x over previous
import jax
import jax.numpy as jnp
from jax.experimental import pallas as pl
from jax.experimental.pallas import tpu as pltpu

_B, _D, _E, _H, _C = 2048, 1024, 16, 256, 100
_CP = 128


def _fused_kernel(x_ref, proto_ref, W1_ref, b1_ref, g_ref, bb_ref,
                  W2_ref, b2_ref, out_ref, w_scratch, x16_scratch):
    e = pl.program_id(0)

    @pl.when(e == 0)
    def _compute_router():
        xf = x_ref[...]
        xn = xf / (jnp.sqrt(jnp.sum(xf * xf, axis=1, keepdims=True)) + 1e-8)
        p = proto_ref[...]
        pn = p / (jnp.sqrt(jnp.sum(p * p, axis=1, keepdims=True)) + 1e-8)
        sims = jnp.dot(xn, pn.T, preferred_element_type=jnp.float32)
        w_scratch[...] = jax.nn.softmax(sims, axis=-1)
        x16_scratch[...] = xf.astype(jnp.bfloat16)

    xb = x16_scratch[...]
    w1 = W1_ref[0].astype(jnp.bfloat16)
    h = jnp.dot(xb, w1, preferred_element_type=jnp.float32) + b1_ref[0]
    mu = jnp.mean(h, axis=-1, keepdims=True)
    var = jnp.mean((h - mu) ** 2, axis=-1, keepdims=True)
    h = (h - mu) / jnp.sqrt(var + 1e-5)
    h = h * g_ref[0] + bb_ref[0]
    h = jax.nn.gelu(h).astype(jnp.bfloat16)
    w2 = W2_ref[0].astype(jnp.bfloat16)
    logits = jnp.dot(h, w2, preferred_element_type=jnp.float32) + b2_ref[0]

    w = w_scratch[...]
    lane = jax.lax.broadcasted_iota(jnp.int32, w.shape, 1)
    w_col = jnp.sum(jnp.where(lane == e, w, 0.0), axis=1, keepdims=True)

    @pl.when(e == 0)
    def _init():
        out_ref[...] = w_col * logits

    @pl.when(e != 0)
    def _acc():
        out_ref[...] += w_col * logits


@jax.jit
def kernel(x, prototypes, W1, b1, ln_g, ln_b, W2, b2):
    W2p = jnp.pad(W2, ((0, 0), (0, 0), (0, _CP - _C)))
    b2p = jnp.pad(b2, ((0, 0), (0, _CP - _C)))
    b1r = b1.reshape(_E, 1, _H)
    gr = ln_g.reshape(_E, 1, _H)
    br = ln_b.reshape(_E, 1, _H)
    b2r = b2p.reshape(_E, 1, _CP)
    out = pl.pallas_call(
        _fused_kernel,
        grid=(_E,),
        in_specs=[
            pl.BlockSpec((_B, _D), lambda e: (0, 0)),
            pl.BlockSpec((_E, _D), lambda e: (0, 0)),
            pl.BlockSpec((1, _D, _H), lambda e: (e, 0, 0)),
            pl.BlockSpec((1, 1, _H), lambda e: (e, 0, 0)),
            pl.BlockSpec((1, 1, _H), lambda e: (e, 0, 0)),
            pl.BlockSpec((1, 1, _H), lambda e: (e, 0, 0)),
            pl.BlockSpec((1, _H, _CP), lambda e: (e, 0, 0)),
            pl.BlockSpec((1, 1, _CP), lambda e: (e, 0, 0)),
        ],
        out_specs=pl.BlockSpec((_B, _CP), lambda e: (0, 0)),
        out_shape=jax.ShapeDtypeStruct((_B, _CP), jnp.float32),
        scratch_shapes=[pltpu.VMEM((_B, _E), jnp.float32),
                        pltpu.VMEM((_B, _D), jnp.bfloat16)],
        compiler_params=pltpu.CompilerParams(
            dimension_semantics=("arbitrary",)),
    )(x, prototypes, W1, b1r, gr, br, W2p, b2r)
    return out[:, :_C]
